# Initial kernel scaffold; baseline (speedup 1.0000x reference)
import jax
import jax.numpy as jnp
from jax.experimental import pallas as pl


def kernel(snp, params, snp_ids, snp_segment_ids, gene_edge_index):
    raise NotImplementedError("write your pallas kernel here")


# speedup vs baseline: 16.2839x; 16.2839x over previous
import functools

import jax
import jax.numpy as jnp
from jax import lax
from jax.experimental import pallas as pl
from jax.experimental.pallas import tpu as pltpu
from jax.experimental.pallas import tpu_sc as plsc

F32 = jnp.float32
I32 = jnp.int32
_BF = jnp.bfloat16

_G = 2000
_GP = 2048
_NPAIR = 8
_NODES_PAD = 20480
_PER_TILE = 640
_AROWS = 2016

def _sc_mesh():
    return plsc.VectorSubcoreMesh(core_axis_name="c", subcore_axis_name="s",
                                  num_cores=2, num_subcores=16)

_KPAT = [2048 * k for k in range(8)] * 2
_VVIDX = list(range(8)) * 2


def _vperm_idx(k):
    lane = lax.iota(I32, 16)
    return (lane + k) & 15


def _vperm(x, idx):
    dnums = lax.GatherDimensionNumbers(
        offset_dims=(), collapsed_slice_dims=(0,), start_index_map=(0,))
    return lax.gather(x, idx[:, None], dnums, slice_sizes=(1,),
                      mode=lax.GatherScatterMode.PROMISE_IN_BOUNDS)

def _s1_body(t_hbm, ids_hbm, seg_hbm, out_hbm,
             ids_v, seg_v, rows_v, p_v, shared):
    c = lax.axis_index("c")
    s = lax.axis_index("s")
    wid = c * 16 + s
    lane = lax.iota(I32, 16)
    zero16 = jnp.zeros((16,), F32)
    vvidx = lane & 7
    takes = [2 * k + (lane >> 3) for k in range(8)]

    def _zrow(i, carry):
        for k in range(8):
            p_v[i, pl.ds(k * 16, 16)] = zero16
        return carry
    lax.fori_loop(0, 128, _zrow, 0)
    pltpu.sync_copy(p_v, shared.at[pl.ds(s * 128, 128)])
    plsc.subcore_barrier()

    pltpu.sync_copy(ids_hbm.at[wid], ids_v)
    pltpu.sync_copy(seg_hbm.at[wid], seg_v)

    def _chunk(ch, carry):
        pltpu.sync_copy(t_hbm.at[ids_v.at[ch]], rows_v)

        def _node(j, carry2):
            u = rows_v[j, pl.ds(0, 16)]
            fv = rows_v[j, pl.ds(16, 16)]
            vv = _vperm(fv, vvidx)
            for k in range(8):
                p_v[j, pl.ds(k * 16, 16)] = _vperm(u, takes[k]) * vv
            return carry2
        lax.fori_loop(0, 128, _node, 0)
        pltpu.sync_copy(p_v, shared.at[seg_v.at[ch]], add=True)
        return carry
    lax.fori_loop(0, 5, _chunk, 0)
    plsc.subcore_barrier()

    pltpu.sync_copy(shared.at[pl.ds(s * 128, 128)], p_v)
    pltpu.sync_copy(p_v, out_hbm.at[c].at[pl.ds(s * 128, 128)])


@jax.jit
def _s1(t, ids3d, seg3d):
    return pl.kernel(
        _s1_body,
        out_type=jax.ShapeDtypeStruct((2, _GP, 128), F32),
        mesh=_sc_mesh(),
        scratch_types=[
            pltpu.VMEM((5, 128), I32),
            pltpu.VMEM((5, 128), I32),
            pltpu.VMEM((128, 128), F32),
            pltpu.VMEM((128, 128), F32),
            pltpu.VMEM_SHARED((_GP, 128), F32),
        ],
    )(t, ids3d, seg3d)


def _s2_body(dst_hbm, src_hbm, out_hbm, p_v, idx_v, cidx_v, dst_v, src_v,
             shared):
    c = lax.axis_index("c")
    s = lax.axis_index("s")
    lane = lax.iota(I32, 16)
    zero16 = jnp.zeros((16,), F32)
    cvecs = [lane + 16 * k for k in range(8)]

    def _idrow(r, carry):
        for k in range(8):
            p_v[r, pl.ds(k * 16, 16)] = jnp.where(cvecs[k] == r, 1.0, 0.0)
        return carry
    lax.fori_loop(0, 128, _idrow, 0)
    pltpu.sync_copy(p_v, shared.at[pl.ds(8080, 128)])

    ntrip = 15 + jnp.where(s < 10, 1, 0)

    for half in range(2):
        base = (2 * c + half) * 504

        def _zrow(i, carry):
            for k in range(8):
                p_v[i, pl.ds(k * 16, 16)] = zero16
            return carry
        lax.fori_loop(0, 128, _zrow, 0)

        def _zacc(i, carry):
            pltpu.sync_copy(p_v.at[pl.ds(0, 72)],
                            shared.at[pl.ds(s * 504 + i * 72, 72)])
            return carry
        lax.fori_loop(0, 7, _zacc, 0)
        pltpu.sync_copy(p_v.at[pl.ds(0, 16)], shared.at[pl.ds(8064, 16)])
        plsc.subcore_barrier()

        def _chunk(i, carry):
            e = s + i * 16
            pltpu.sync_copy(dst_hbm.at[e], dst_v)
            pltpu.sync_copy(src_hbm.at[e], src_v)

            def _idx(t, carry2):
                d16 = dst_v[0, pl.ds(t * 16, 16)]
                s16 = src_v[0, pl.ds(t * 16, 16)]
                loc = d16 - base
                valid = (loc >= 0) & (loc < 504)
                idx_v[0, pl.ds(t * 16, 16)] = jnp.where(
                    valid, loc * 16 + (s16 >> 7), 8064)
                cidx_v[0, pl.ds(t * 16, 16)] = 8080 + (s16 & 127)
                return carry2
            lax.fori_loop(0, 8, _idx, 0)

            pltpu.sync_copy(shared.at[cidx_v.at[0]], p_v)
            pltpu.sync_copy(p_v, shared.at[idx_v.at[0]], add=True)
            return carry
        lax.fori_loop(0, ntrip, _chunk, 0)
        plsc.subcore_barrier()

        def _wb(i, carry):
            pltpu.sync_copy(shared.at[pl.ds(s * 504 + i * 72, 72)],
                            p_v.at[pl.ds(0, 72)])
            pltpu.sync_copy(p_v.at[pl.ds(0, 72)],
                            out_hbm.at[2 * c + half].at[pl.ds(s * 504 + i * 72,
                                                              72)])
            return carry
        lax.fori_loop(0, 7, _wb, 0)
        plsc.subcore_barrier()


@jax.jit
def _s2(dst3d, src3d):
    return pl.kernel(
        _s2_body,
        out_type=jax.ShapeDtypeStruct((4, 8064, 128), F32),
        mesh=_sc_mesh(),
        scratch_types=[
            pltpu.VMEM((128, 128), F32),
            pltpu.VMEM((1, 128), I32),
            pltpu.VMEM((1, 128), I32),
            pltpu.VMEM((1, 128), I32),
            pltpu.VMEM((1, 128), I32),
            pltpu.VMEM_SHARED((8208, 128), F32),
        ],
    )(dst3d, src3d)


def _mm_body(x_ref, w_ref, o_ref):
    o_ref[...] = jnp.dot(x_ref[...], w_ref[...], preferred_element_type=F32)


def _pmm(x, w):
    return pl.pallas_call(
        _mm_body,
        out_shape=jax.ShapeDtypeStruct((x.shape[0], w.shape[1]), F32),
    )(x, w)


def _agg_body(ah_ref, al_ref, h_ref, o_ref):
    ah = ah_ref[...]
    al = al_ref[...]
    h = h_ref[...]
    hh = h.astype(_BF)
    hl = (h - hh.astype(F32)).astype(_BF)
    o_ref[...] = (jnp.dot(ah, hh, preferred_element_type=F32)
                  + jnp.dot(ah, hl, preferred_element_type=F32)
                  + jnp.dot(al, hh, preferred_element_type=F32))


def _pagg(ah, al, h2d):
    return pl.pallas_call(
        _agg_body,
        out_shape=jax.ShapeDtypeStruct((ah.shape[0], h2d.shape[1]), F32),
    )(ah, al, h2d)


def kernel(snp, params, snp_ids, snp_segment_ids, gene_edge_index):
    p = params

    n_snps = snp.shape[1]
    t = jnp.concatenate(
        [snp.T, p["filters"].T, jnp.zeros((n_snps, 104), F32)], axis=1)
    npad = _NODES_PAD - snp_ids.shape[0]
    ids3d = jnp.concatenate(
        [snp_ids.astype(I32), jnp.zeros((npad,), I32)]).reshape(32, 5, 128)
    seg3d = jnp.concatenate(
        [snp_segment_ids.astype(I32),
         jnp.full((npad,), _GP - 1, I32)]).reshape(32, 5, 128)
    dst3d = gene_edge_index[1].astype(I32).reshape(250, 1, 128)
    src3d = gene_edge_index[0].astype(I32).reshape(250, 1, 128)

    p2 = _s1(t, ids3d, seg3d)
    a = _s2(dst3d, src3d).reshape(_AROWS, _GP)[:_G, :_G]
    ah = a.astype(_BF)
    al = (a - ah.astype(F32)).astype(_BF)

    h0 = (p2[0] + p2[1])[:_G].reshape(_G * 16, 8)

    def bn(x, g, b):
        m = jnp.mean(x, axis=0)
        v = jnp.var(x, axis=0)
        return (x - m) / jnp.sqrt(v + 1e-5) * g + b

    h = _pmm(h0, p["ge_W1"]) + p["ge_b1"]
    h = jax.nn.relu(bn(h, p["ge_g1"], p["ge_be1"]))
    h = _pmm(h, p["ge_W2"]) + p["ge_b2"]

    for i in range(2):
        agg2d = _pagg(ah, al, h.reshape(_G, 16 * 64))
        rst = (h.reshape(_G, 16 * 64) + agg2d).reshape(_G * 16, 64)
        m = _pmm(rst, p["gin%d_W1" % i]) + p["gin%d_b1" % i]
        m = jax.nn.relu(bn(m, p["gin%d_g1" % i], p["gin%d_be1" % i]))
        m = _pmm(m, p["gin%d_W2" % i]) + p["gin%d_b2" % i]
        h = jax.nn.relu(bn(m, p["obn%d_g" % i], p["obn%d_b" % i]))

    keys_ = _pmm(h, p["key_W"]) + p["key_b"]
    w = jax.nn.sigmoid(_pmm(keys_, p["q_W"]))
    v = _pmm(h, p["val_W"]) + p["val_b"]
    g_h = (w * v).reshape(_G, 16, 64).sum(axis=0)
    weights = w.reshape(_G, 16).T

    q = _pmm(g_h, p["p_W1"]) + p["p_b1"]
    q = jax.nn.relu(bn(q, p["p_g1"], p["p_be1"]))
    q = _pmm(q, p["p_W2"]) + p["p_b2"]
    q = jax.nn.relu(bn(q, p["p_g2"], p["p_be2"]))
    preds = _pmm(q, p["p_W3"]) + p["p_b3"]
    return (preds, p["filters"], weights)
